# baseline (speedup 1.0000x reference)
import functools

import jax
import jax.numpy as jnp
from jax import lax
from jax.experimental import pallas as pl
from jax.experimental.pallas import tpu as pltpu
from jax.experimental.pallas import tpu_sc as plsc

N_NODES = 10000
NP = 10240
D = 32
E = 320000
NC, NS, B = 2, 16, 128
BD = 128
EP = 327680
BLOCKS = EP // (NC * NS * B)
BLOCKS_D = EP // (NC * NS * BD)
ROWS_PT = NP // NS
TRASH = N_NODES

_MESH = plsc.VectorSubcoreMesh(
    core_axis_name="c", subcore_axis_name="s", num_cores=NC, num_subcores=NS)


@functools.partial(
    pl.kernel,
    out_type=jax.ShapeDtypeStruct((NC, NP), jnp.float32),
    mesh=_MESH,
    scratch_types=[
        pltpu.VMEM_SHARED((NP,), jnp.float32),
        pltpu.VMEM((BLOCKS_D, BD), jnp.int32),
        pltpu.VMEM((BD,), jnp.float32),
    ],
)
def _sc_degree(dst_hbm, zeros_hbm, out_hbm, acc_sh, didx, ones_v):
    c = lax.axis_index("c")
    s = lax.axis_index("s")
    r0 = s * ROWS_PT
    pltpu.sync_copy(zeros_hbm.at[pl.ds(r0, ROWS_PT)], acc_sh.at[pl.ds(r0, ROWS_PT)])
    pltpu.sync_copy(dst_hbm.at[pl.ds((c * NS + s) * BLOCKS_D, BLOCKS_D)], didx)
    for i in range(BD // 16):
        ones_v[pl.ds(i * 16, 16)] = jnp.full((16,), 1.0, jnp.float32)
    plsc.subcore_barrier()

    def body(j, carry):
        pltpu.sync_copy(ones_v, acc_sh.at[didx.at[j]], add=True)
        return carry

    lax.fori_loop(0, BLOCKS_D, body, 0)
    plsc.subcore_barrier()
    pltpu.sync_copy(acc_sh.at[pl.ds(r0, ROWS_PT)], out_hbm.at[c].at[pl.ds(r0, ROWS_PT)])


@functools.partial(
    pl.kernel,
    out_type=jax.ShapeDtypeStruct((NC, NP, D), jnp.float32),
    mesh=_MESH,
    scratch_types=[
        pltpu.VMEM_SHARED((NP, D), jnp.float32),
        pltpu.VMEM_SHARED((NP, D), jnp.float32),
        pltpu.VMEM((BLOCKS, B), jnp.int32),
        pltpu.VMEM((BLOCKS, B), jnp.int32),
        pltpu.VMEM((B, D), jnp.float32),
        pltpu.VMEM((B, D), jnp.float32),
        pltpu.SemaphoreType.DMA,
        pltpu.SemaphoreType.DMA,
    ],
    compiler_params=pltpu.CompilerParams(use_tc_tiling_on_sc=False),
)
def _sc_aggregate(g_hbm, src_hbm, dst_hbm, zeros_hbm, out_hbm,
                  acc_sh, g_spm, sidx, didx, rows0, rows1, sem0, sem1):
    c = lax.axis_index("c")
    s = lax.axis_index("s")
    r0 = s * ROWS_PT
    pltpu.sync_copy(zeros_hbm.at[pl.ds(r0, ROWS_PT)], acc_sh.at[pl.ds(r0, ROWS_PT)])
    pltpu.sync_copy(g_hbm.at[pl.ds(r0, ROWS_PT)], g_spm.at[pl.ds(r0, ROWS_PT)])
    blk0 = (c * NS + s) * BLOCKS
    pltpu.sync_copy(src_hbm.at[pl.ds(blk0, BLOCKS)], sidx)
    pltpu.sync_copy(dst_hbm.at[pl.ds(blk0, BLOCKS)], didx)
    plsc.subcore_barrier()

    pltpu.async_copy(g_spm.at[sidx.at[0]], rows0, sem0)

    def body(j, carry):
        b0 = j * 2
        b1 = b0 + 1
        pltpu.make_async_copy(g_spm.at[sidx.at[b0]], rows0, sem0).wait()
        pltpu.async_copy(g_spm.at[sidx.at[b1]], rows1, sem1)
        pltpu.sync_copy(rows0, acc_sh.at[didx.at[b0]], add=True)
        pltpu.make_async_copy(g_spm.at[sidx.at[b1]], rows1, sem1).wait()

        @pl.when(b1 + 1 < BLOCKS)
        def _():
            pltpu.async_copy(g_spm.at[sidx.at[b1 + 1]], rows0, sem0)

        pltpu.sync_copy(rows1, acc_sh.at[didx.at[b1]], add=True)
        return carry

    lax.fori_loop(0, BLOCKS // 2, body, 0)
    plsc.subcore_barrier()
    pltpu.sync_copy(acc_sh.at[pl.ds(r0, ROWS_PT)], out_hbm.at[c].at[pl.ds(r0, ROWS_PT)])


CH = 128
NCH = ROWS_PT // CH


@functools.partial(
    pl.kernel,
    out_type=[jax.ShapeDtypeStruct((NC, NP, D), jnp.float32),
              jax.ShapeDtypeStruct((NP, D), jnp.float32)],
    mesh=_MESH,
    scratch_types=[
        pltpu.VMEM_SHARED((NP, D), jnp.float32),
        pltpu.VMEM_SHARED((NP, D), jnp.float32),
        pltpu.VMEM((BLOCKS, B), jnp.int32),
        pltpu.VMEM((BLOCKS, B), jnp.int32),
        pltpu.VMEM((B, D), jnp.float32),
        pltpu.VMEM((B, D), jnp.float32),
        pltpu.VMEM((CH, D), jnp.float32),
        pltpu.VMEM((CH, D), jnp.float32),
        pltpu.VMEM((CH, D), jnp.float32),
        pltpu.VMEM((CH, 16), jnp.float32),
        pltpu.VMEM((8, D), jnp.float32),
        pltpu.SemaphoreType.DMA,
        pltpu.SemaphoreType.DMA,
    ],
    compiler_params=pltpu.CompilerParams(use_tc_tiling_on_sc=False),
)
def _sc_fused_last(a_hbm, g3_hbm, dinvb_hbm, b_hbm, src_hbm, dst_hbm, zeros_hbm,
                   out_hbm, g4_hbm, acc_sh, g_spm, sidx, didx, rows0, rows1,
                   a0c, a1c, gc, dvc, bc, sem0, sem1):
    c = lax.axis_index("c")
    s = lax.axis_index("s")
    r0 = s * ROWS_PT
    pltpu.sync_copy(zeros_hbm.at[pl.ds(r0, ROWS_PT)], acc_sh.at[pl.ds(r0, ROWS_PT)])
    blk0 = (c * NS + s) * BLOCKS
    pltpu.sync_copy(src_hbm.at[pl.ds(blk0, BLOCKS)], sidx)
    pltpu.sync_copy(dst_hbm.at[pl.ds(blk0, BLOCKS)], didx)
    pltpu.sync_copy(b_hbm, bc)
    blo = bc[0, pl.ds(0, 16)]
    bhi = bc[0, pl.ds(16, 16)]

    for ci in range(NCH):
        rc = r0 + ci * CH
        pltpu.sync_copy(a_hbm.at[0].at[pl.ds(rc, CH)], a0c)
        pltpu.sync_copy(a_hbm.at[1].at[pl.ds(rc, CH)], a1c)
        pltpu.sync_copy(g3_hbm.at[pl.ds(rc, CH)], gc)
        pltpu.sync_copy(dinvb_hbm.at[pl.ds(rc, CH)], dvc)

        def row_body(r, carry):
            vd = dvc[r, :]
            lo = a0c[r, pl.ds(0, 16)] + a1c[r, pl.ds(0, 16)] + gc[r, pl.ds(0, 16)]
            hi = a0c[r, pl.ds(16, 16)] + a1c[r, pl.ds(16, 16)] + gc[r, pl.ds(16, 16)]
            a0c[r, pl.ds(0, 16)] = jnp.maximum(lo * vd + blo, 0.0) * vd
            a0c[r, pl.ds(16, 16)] = jnp.maximum(hi * vd + bhi, 0.0) * vd
            return carry

        lax.fori_loop(0, CH, row_body, 0)
        pltpu.sync_copy(a0c, g_spm.at[pl.ds(rc, CH)])

        @pl.when(c == 0)
        def _():
            pltpu.sync_copy(a0c, g4_hbm.at[pl.ds(rc, CH)])

    plsc.subcore_barrier()

    pltpu.async_copy(g_spm.at[sidx.at[0]], rows0, sem0)

    def body(j, carry):
        b0 = j * 2
        b1 = b0 + 1
        pltpu.make_async_copy(g_spm.at[sidx.at[b0]], rows0, sem0).wait()
        pltpu.async_copy(g_spm.at[sidx.at[b1]], rows1, sem1)
        pltpu.sync_copy(rows0, acc_sh.at[didx.at[b0]], add=True)
        pltpu.make_async_copy(g_spm.at[sidx.at[b1]], rows1, sem1).wait()

        @pl.when(b1 + 1 < BLOCKS)
        def _():
            pltpu.async_copy(g_spm.at[sidx.at[b1 + 1]], rows0, sem0)

        pltpu.sync_copy(rows1, acc_sh.at[didx.at[b1]], add=True)
        return carry

    lax.fori_loop(0, BLOCKS // 2, body, 0)
    plsc.subcore_barrier()
    pltpu.sync_copy(acc_sh.at[pl.ds(r0, ROWS_PT)], out_hbm.at[c].at[pl.ds(r0, ROWS_PT)])


def _tc_matmul1(x_ref, w1_ref, h_ref):
    h_ref[...] = jnp.dot(x_ref[...], w1_ref[...],
                         preferred_element_type=jnp.float32)


def _tc_head(degpt_ref, h_ref, dinv_ref, dinvb_ref, g1_ref):
    dp = degpt_ref[...]
    deg = dp[:, 0:1] + dp[:, 1:2] + 1.0
    dinv = lax.rsqrt(deg)
    dinv_ref[...] = dinv
    dinvb_ref[...] = jnp.broadcast_to(dinv, (NP, 16))
    g1_ref[...] = h_ref[...] * dinv


def _tc_mid(a0_ref, a1_ref, g_ref, dinv_ref, b_ref, w_ref, gn_ref):
    dinv = dinv_ref[...]
    agg = (a0_ref[...] + a1_ref[...] + g_ref[...]) * dinv + b_ref[...]
    out = jnp.maximum(agg, 0.0)
    gn_ref[...] = jnp.dot(out, w_ref[...], preferred_element_type=jnp.float32) * dinv


def _tc_tail(a0_ref, a1_ref, g_ref, dinv_ref, w4_ref, b4_ref, out_ref):
    a = (a0_ref[...] + a1_ref[...] + g_ref[...]) * dinv_ref[...]
    logits = jnp.dot(a, w4_ref[...], preferred_element_type=jnp.float32) + b4_ref[...]
    m = jnp.max(logits, axis=1, keepdims=True)
    z = logits - m
    out_ref[...] = z - jnp.log(jnp.sum(jnp.exp(z), axis=1, keepdims=True))


def _call(body, out_shapes, *args):
    return pl.pallas_call(
        body,
        out_shape=[jax.ShapeDtypeStruct(s, jnp.float32) for s in out_shapes],
    )(*args)


@jax.jit
def kernel(x, edge_index, W1, b1, W2, b2, W3, b3, W4, b4):
    src = edge_index[0].astype(jnp.int32)
    dst = edge_index[1].astype(jnp.int32)
    pad = EP - E
    src_p = jnp.concatenate([src, jnp.zeros((pad,), jnp.int32)])
    dst_p = jnp.concatenate([dst, jnp.full((pad,), TRASH, jnp.int32)])
    src2d = src_p.reshape(EP // B, B)
    dst2d = dst_p.reshape(EP // B, B)
    zeros1 = jnp.zeros((NP,), jnp.float32)
    zeros2 = jnp.zeros((NP, D), jnp.float32)

    degp = _sc_degree(dst_p.reshape(EP // BD, BD), zeros1)
    (h1,) = _call(_tc_matmul1, [(N_NODES, D)], x, W1)
    h1_p = jnp.pad(h1, ((0, NP - N_NODES), (0, 0)))
    dinv, dinvb, g = _call(_tc_head, [(NP, 1), (NP, 16), (NP, D)], degp.T, h1_p)

    for bk, wn in ((b1, W2), (b2, W3)):
        ap = _sc_aggregate(g, src2d, dst2d, zeros2)
        (g,) = _call(_tc_mid, [(NP, D)], ap[0], ap[1], g, dinv,
                     bk.reshape(1, D), wn)

    ap = _sc_aggregate(g, src2d, dst2d, zeros2)
    b3b = jnp.broadcast_to(b3.reshape(1, D), (8, D))
    ap4, g4 = _sc_fused_last(ap, g, dinvb, b3b, src2d, dst2d, zeros2)
    (out,) = _call(_tc_tail, [(NP, 2)], ap4[0], ap4[1], g4, dinv, W4,
                   b4.reshape(1, 2))
    return out[:N_NODES]


# speedup vs baseline: 1.0582x; 1.0582x over previous
import functools

import jax
import jax.numpy as jnp
from jax import lax
from jax.experimental import pallas as pl
from jax.experimental.pallas import tpu as pltpu
from jax.experimental.pallas import tpu_sc as plsc

N_NODES = 10000
NP = 10240
D = 32
E = 320000
NC, NS, B = 2, 16, 256
BD = 128
EP = 327680
BLOCKS = EP // (NC * NS * B)
BLOCKS_D = EP // (NC * NS * BD)
ROWS_PT = NP // NS
TRASH = N_NODES

_MESH = plsc.VectorSubcoreMesh(
    core_axis_name="c", subcore_axis_name="s", num_cores=NC, num_subcores=NS)


@functools.partial(
    pl.kernel,
    out_type=jax.ShapeDtypeStruct((NC, NP), jnp.float32),
    mesh=_MESH,
    scratch_types=[
        pltpu.VMEM_SHARED((NP,), jnp.float32),
        pltpu.VMEM((BLOCKS_D, BD), jnp.int32),
        pltpu.VMEM((BD,), jnp.float32),
    ],
)
def _sc_degree(dst_hbm, zeros_hbm, out_hbm, acc_sh, didx, ones_v):
    c = lax.axis_index("c")
    s = lax.axis_index("s")
    r0 = s * ROWS_PT
    pltpu.sync_copy(zeros_hbm.at[pl.ds(r0, ROWS_PT)], acc_sh.at[pl.ds(r0, ROWS_PT)])
    pltpu.sync_copy(dst_hbm.at[pl.ds((c * NS + s) * BLOCKS_D, BLOCKS_D)], didx)
    for i in range(BD // 16):
        ones_v[pl.ds(i * 16, 16)] = jnp.full((16,), 1.0, jnp.float32)
    plsc.subcore_barrier()

    def body(j, carry):
        pltpu.sync_copy(ones_v, acc_sh.at[didx.at[j]], add=True)
        return carry

    lax.fori_loop(0, BLOCKS_D, body, 0)
    plsc.subcore_barrier()
    pltpu.sync_copy(acc_sh.at[pl.ds(r0, ROWS_PT)], out_hbm.at[c].at[pl.ds(r0, ROWS_PT)])


@functools.partial(
    pl.kernel,
    out_type=jax.ShapeDtypeStruct((NC, NP, D), jnp.float32),
    mesh=_MESH,
    scratch_types=[
        pltpu.VMEM_SHARED((NP, D), jnp.float32),
        pltpu.VMEM_SHARED((NP, D), jnp.float32),
        pltpu.VMEM((BLOCKS, B), jnp.int32),
        pltpu.VMEM((BLOCKS, B), jnp.int32),
        pltpu.VMEM((B, D), jnp.float32),
        pltpu.VMEM((B, D), jnp.float32),
        pltpu.SemaphoreType.DMA,
        pltpu.SemaphoreType.DMA,
        pltpu.SemaphoreType.DMA,
        pltpu.SemaphoreType.DMA,
    ],
    compiler_params=pltpu.CompilerParams(use_tc_tiling_on_sc=False),
)
def _sc_aggregate(g_hbm, src_hbm, dst_hbm, zeros_hbm, out_hbm,
                  acc_sh, g_spm, sidx, didx, rows0, rows1, sem0, sem1,
                  sem2, sem3):
    c = lax.axis_index("c")
    s = lax.axis_index("s")
    r0 = s * ROWS_PT
    blk0 = (c * NS + s) * BLOCKS
    pltpu.async_copy(zeros_hbm.at[pl.ds(r0, ROWS_PT)], acc_sh.at[pl.ds(r0, ROWS_PT)], sem0)
    pltpu.async_copy(g_hbm.at[pl.ds(r0, ROWS_PT)], g_spm.at[pl.ds(r0, ROWS_PT)], sem1)
    pltpu.async_copy(src_hbm.at[pl.ds(blk0, BLOCKS)], sidx, sem2)
    pltpu.async_copy(dst_hbm.at[pl.ds(blk0, BLOCKS)], didx, sem3)
    pltpu.make_async_copy(zeros_hbm.at[pl.ds(r0, ROWS_PT)], acc_sh.at[pl.ds(r0, ROWS_PT)], sem0).wait()
    pltpu.make_async_copy(g_hbm.at[pl.ds(r0, ROWS_PT)], g_spm.at[pl.ds(r0, ROWS_PT)], sem1).wait()
    pltpu.make_async_copy(src_hbm.at[pl.ds(blk0, BLOCKS)], sidx, sem2).wait()
    pltpu.make_async_copy(dst_hbm.at[pl.ds(blk0, BLOCKS)], didx, sem3).wait()
    plsc.subcore_barrier()

    pltpu.async_copy(g_spm.at[sidx.at[0]], rows0, sem0)

    def body(j, carry):
        b0 = j * 2
        b1 = b0 + 1
        pltpu.make_async_copy(g_spm.at[sidx.at[b0]], rows0, sem0).wait()
        pltpu.async_copy(g_spm.at[sidx.at[b1]], rows1, sem1)
        pltpu.sync_copy(rows0, acc_sh.at[didx.at[b0]], add=True)
        pltpu.make_async_copy(g_spm.at[sidx.at[b1]], rows1, sem1).wait()

        @pl.when(b1 + 1 < BLOCKS)
        def _():
            pltpu.async_copy(g_spm.at[sidx.at[b1 + 1]], rows0, sem0)

        pltpu.sync_copy(rows1, acc_sh.at[didx.at[b1]], add=True)
        return carry

    lax.fori_loop(0, BLOCKS // 2, body, 0)
    plsc.subcore_barrier()
    pltpu.sync_copy(acc_sh.at[pl.ds(r0, ROWS_PT)], out_hbm.at[c].at[pl.ds(r0, ROWS_PT)])


CH = 128
NCH = ROWS_PT // CH


@functools.partial(
    pl.kernel,
    out_type=[jax.ShapeDtypeStruct((NC, NP, D), jnp.float32),
              jax.ShapeDtypeStruct((NP, D), jnp.float32)],
    mesh=_MESH,
    scratch_types=[
        pltpu.VMEM_SHARED((NP, D), jnp.float32),
        pltpu.VMEM_SHARED((NP, D), jnp.float32),
        pltpu.VMEM((BLOCKS, B), jnp.int32),
        pltpu.VMEM((BLOCKS, B), jnp.int32),
        pltpu.VMEM((B, D), jnp.float32),
        pltpu.VMEM((B, D), jnp.float32),
        pltpu.VMEM((CH, D), jnp.float32),
        pltpu.VMEM((CH, D), jnp.float32),
        pltpu.VMEM((CH, D), jnp.float32),
        pltpu.VMEM((CH, 16), jnp.float32),
        pltpu.VMEM((8, D), jnp.float32),
        pltpu.SemaphoreType.DMA,
        pltpu.SemaphoreType.DMA,
        pltpu.SemaphoreType.DMA,
        pltpu.SemaphoreType.DMA,
    ],
    compiler_params=pltpu.CompilerParams(use_tc_tiling_on_sc=False),
)
def _sc_fused_last(a_hbm, g3_hbm, dinvb_hbm, b_hbm, src_hbm, dst_hbm, zeros_hbm,
                   out_hbm, g4_hbm, acc_sh, g_spm, sidx, didx, rows0, rows1,
                   a0c, a1c, gc, dvc, bc, sem0, sem1, sem2, sem3):
    c = lax.axis_index("c")
    s = lax.axis_index("s")
    r0 = s * ROWS_PT
    blk0 = (c * NS + s) * BLOCKS
    pltpu.async_copy(zeros_hbm.at[pl.ds(r0, ROWS_PT)], acc_sh.at[pl.ds(r0, ROWS_PT)], sem0)
    pltpu.async_copy(src_hbm.at[pl.ds(blk0, BLOCKS)], sidx, sem2)
    pltpu.async_copy(dst_hbm.at[pl.ds(blk0, BLOCKS)], didx, sem3)
    pltpu.sync_copy(b_hbm, bc)
    pltpu.make_async_copy(zeros_hbm.at[pl.ds(r0, ROWS_PT)], acc_sh.at[pl.ds(r0, ROWS_PT)], sem0).wait()
    pltpu.make_async_copy(src_hbm.at[pl.ds(blk0, BLOCKS)], sidx, sem2).wait()
    pltpu.make_async_copy(dst_hbm.at[pl.ds(blk0, BLOCKS)], didx, sem3).wait()
    blo = bc[0, pl.ds(0, 16)]
    bhi = bc[0, pl.ds(16, 16)]

    for ci in range(NCH):
        rc = r0 + ci * CH
        pltpu.async_copy(a_hbm.at[0].at[pl.ds(rc, CH)], a0c, sem0)
        pltpu.async_copy(a_hbm.at[1].at[pl.ds(rc, CH)], a1c, sem1)
        pltpu.async_copy(g3_hbm.at[pl.ds(rc, CH)], gc, sem2)
        pltpu.async_copy(dinvb_hbm.at[pl.ds(rc, CH)], dvc, sem3)
        pltpu.make_async_copy(a_hbm.at[0].at[pl.ds(rc, CH)], a0c, sem0).wait()
        pltpu.make_async_copy(a_hbm.at[1].at[pl.ds(rc, CH)], a1c, sem1).wait()
        pltpu.make_async_copy(g3_hbm.at[pl.ds(rc, CH)], gc, sem2).wait()
        pltpu.make_async_copy(dinvb_hbm.at[pl.ds(rc, CH)], dvc, sem3).wait()

        def row_body(r, carry):
            vd = dvc[r, :]
            lo = a0c[r, pl.ds(0, 16)] + a1c[r, pl.ds(0, 16)] + gc[r, pl.ds(0, 16)]
            hi = a0c[r, pl.ds(16, 16)] + a1c[r, pl.ds(16, 16)] + gc[r, pl.ds(16, 16)]
            a0c[r, pl.ds(0, 16)] = jnp.maximum(lo * vd + blo, 0.0) * vd
            a0c[r, pl.ds(16, 16)] = jnp.maximum(hi * vd + bhi, 0.0) * vd
            return carry

        lax.fori_loop(0, CH, row_body, 0)
        pltpu.sync_copy(a0c, g_spm.at[pl.ds(rc, CH)])

        @pl.when(c == 0)
        def _():
            pltpu.sync_copy(a0c, g4_hbm.at[pl.ds(rc, CH)])

    plsc.subcore_barrier()

    pltpu.async_copy(g_spm.at[sidx.at[0]], rows0, sem0)

    def body(j, carry):
        b0 = j * 2
        b1 = b0 + 1
        pltpu.make_async_copy(g_spm.at[sidx.at[b0]], rows0, sem0).wait()
        pltpu.async_copy(g_spm.at[sidx.at[b1]], rows1, sem1)
        pltpu.sync_copy(rows0, acc_sh.at[didx.at[b0]], add=True)
        pltpu.make_async_copy(g_spm.at[sidx.at[b1]], rows1, sem1).wait()

        @pl.when(b1 + 1 < BLOCKS)
        def _():
            pltpu.async_copy(g_spm.at[sidx.at[b1 + 1]], rows0, sem0)

        pltpu.sync_copy(rows1, acc_sh.at[didx.at[b1]], add=True)
        return carry

    lax.fori_loop(0, BLOCKS // 2, body, 0)
    plsc.subcore_barrier()
    pltpu.sync_copy(acc_sh.at[pl.ds(r0, ROWS_PT)], out_hbm.at[c].at[pl.ds(r0, ROWS_PT)])


def _tc_matmul1(x_ref, w1_ref, h_ref):
    h_ref[...] = jnp.dot(x_ref[...], w1_ref[...],
                         preferred_element_type=jnp.float32)


def _tc_head(degpt_ref, h_ref, dinv_ref, dinvb_ref, g1_ref):
    dp = degpt_ref[...]
    deg = dp[:, 0:1] + dp[:, 1:2] + 1.0
    dinv = lax.rsqrt(deg)
    dinv_ref[...] = dinv
    dinvb_ref[...] = jnp.broadcast_to(dinv, (NP, 16))
    g1_ref[...] = h_ref[...] * dinv


def _tc_mid(a0_ref, a1_ref, g_ref, dinv_ref, b_ref, w_ref, gn_ref):
    dinv = dinv_ref[...]
    agg = (a0_ref[...] + a1_ref[...] + g_ref[...]) * dinv + b_ref[...]
    out = jnp.maximum(agg, 0.0)
    gn_ref[...] = jnp.dot(out, w_ref[...], preferred_element_type=jnp.float32) * dinv


def _tc_tail(a0_ref, a1_ref, g_ref, dinv_ref, w4_ref, b4_ref, out_ref):
    a = (a0_ref[...] + a1_ref[...] + g_ref[...]) * dinv_ref[...]
    logits = jnp.dot(a, w4_ref[...], preferred_element_type=jnp.float32) + b4_ref[...]
    m = jnp.max(logits, axis=1, keepdims=True)
    z = logits - m
    out_ref[...] = z - jnp.log(jnp.sum(jnp.exp(z), axis=1, keepdims=True))


def _call(body, out_shapes, *args):
    return pl.pallas_call(
        body,
        out_shape=[jax.ShapeDtypeStruct(s, jnp.float32) for s in out_shapes],
    )(*args)


@jax.jit
def kernel(x, edge_index, W1, b1, W2, b2, W3, b3, W4, b4):
    src = edge_index[0].astype(jnp.int32)
    dst = edge_index[1].astype(jnp.int32)
    pad = EP - E
    src_p = jnp.concatenate([src, jnp.zeros((pad,), jnp.int32)])
    dst_p = jnp.concatenate([dst, jnp.full((pad,), TRASH, jnp.int32)])
    src2d = src_p.reshape(EP // B, B)
    dst2d = dst_p.reshape(EP // B, B)
    zeros1 = jnp.zeros((NP,), jnp.float32)
    zeros2 = jnp.zeros((NP, D), jnp.float32)

    degp = _sc_degree(dst_p.reshape(EP // BD, BD), zeros1)
    (h1,) = _call(_tc_matmul1, [(N_NODES, D)], x, W1)
    h1_p = jnp.pad(h1, ((0, NP - N_NODES), (0, 0)))
    dinv, dinvb, g = _call(_tc_head, [(NP, 1), (NP, 16), (NP, D)], degp.T, h1_p)

    for bk, wn in ((b1, W2), (b2, W3)):
        ap = _sc_aggregate(g, src2d, dst2d, zeros2)
        (g,) = _call(_tc_mid, [(NP, D)], ap[0], ap[1], g, dinv,
                     bk.reshape(1, D), wn)

    ap = _sc_aggregate(g, src2d, dst2d, zeros2)
    b3b = jnp.broadcast_to(b3.reshape(1, D), (8, D))
    ap4, g4 = _sc_fused_last(ap, g, dinvb, b3b, src2d, dst2d, zeros2)
    (out,) = _call(_tc_tail, [(NP, 2)], ap4[0], ap4[1], g4, dinv, W4,
                   b4.reshape(1, 2))
    return out[:N_NODES]


# speedup vs baseline: 1.0601x; 1.0018x over previous
import functools

import jax
import jax.numpy as jnp
from jax import lax
from jax.experimental import pallas as pl
from jax.experimental.pallas import tpu as pltpu
from jax.experimental.pallas import tpu_sc as plsc

N_NODES = 10000
NP = 10240
D = 32
E = 320000
NC, NS, B = 2, 16, 256
BD = 128
EP = 327680
BLOCKS = EP // (NC * NS * B)
BLOCKS_D = EP // (NC * NS * BD)
ROWS_PT = NP // NS
TRASH = N_NODES

_MESH = plsc.VectorSubcoreMesh(
    core_axis_name="c", subcore_axis_name="s", num_cores=NC, num_subcores=NS)


@functools.partial(
    pl.kernel,
    out_type=jax.ShapeDtypeStruct((NC, NP), jnp.float32),
    mesh=_MESH,
    scratch_types=[
        pltpu.VMEM_SHARED((NP,), jnp.float32),
        pltpu.VMEM((BLOCKS_D, BD), jnp.int32),
        pltpu.VMEM((BD,), jnp.float32),
        pltpu.SemaphoreType.DMA,
        pltpu.SemaphoreType.DMA,
    ],
)
def _sc_degree(dst_hbm, zeros_hbm, out_hbm, acc_sh, didx, ones_v, sem0, sem1):
    c = lax.axis_index("c")
    s = lax.axis_index("s")
    r0 = s * ROWS_PT
    blk0 = (c * NS + s) * BLOCKS_D
    pltpu.async_copy(zeros_hbm.at[pl.ds(r0, ROWS_PT)], acc_sh.at[pl.ds(r0, ROWS_PT)], sem0)
    pltpu.async_copy(dst_hbm.at[pl.ds(blk0, BLOCKS_D)], didx, sem1)
    for i in range(BD // 16):
        ones_v[pl.ds(i * 16, 16)] = jnp.full((16,), 1.0, jnp.float32)
    pltpu.make_async_copy(zeros_hbm.at[pl.ds(r0, ROWS_PT)], acc_sh.at[pl.ds(r0, ROWS_PT)], sem0).wait()
    pltpu.make_async_copy(dst_hbm.at[pl.ds(blk0, BLOCKS_D)], didx, sem1).wait()
    plsc.subcore_barrier()

    def body(j, carry):
        pltpu.sync_copy(ones_v, acc_sh.at[didx.at[j]], add=True)
        return carry

    lax.fori_loop(0, BLOCKS_D, body, 0)
    plsc.subcore_barrier()
    pltpu.sync_copy(acc_sh.at[pl.ds(r0, ROWS_PT)], out_hbm.at[c].at[pl.ds(r0, ROWS_PT)])


@functools.partial(
    pl.kernel,
    out_type=jax.ShapeDtypeStruct((NC, NP, D), jnp.float32),
    mesh=_MESH,
    scratch_types=[
        pltpu.VMEM_SHARED((NP, D), jnp.float32),
        pltpu.VMEM_SHARED((NP, D), jnp.float32),
        pltpu.VMEM((BLOCKS, B), jnp.int32),
        pltpu.VMEM((BLOCKS, B), jnp.int32),
        pltpu.VMEM((B, D), jnp.float32),
        pltpu.VMEM((B, D), jnp.float32),
        pltpu.SemaphoreType.DMA,
        pltpu.SemaphoreType.DMA,
        pltpu.SemaphoreType.DMA,
        pltpu.SemaphoreType.DMA,
    ],
    compiler_params=pltpu.CompilerParams(use_tc_tiling_on_sc=False),
)
def _sc_aggregate(g_hbm, src_hbm, dst_hbm, zeros_hbm, out_hbm,
                  acc_sh, g_spm, sidx, didx, rows0, rows1, sem0, sem1,
                  sem2, sem3):
    c = lax.axis_index("c")
    s = lax.axis_index("s")
    r0 = s * ROWS_PT
    blk0 = (c * NS + s) * BLOCKS
    pltpu.async_copy(zeros_hbm.at[pl.ds(r0, ROWS_PT)], acc_sh.at[pl.ds(r0, ROWS_PT)], sem0)
    pltpu.async_copy(g_hbm.at[pl.ds(r0, ROWS_PT)], g_spm.at[pl.ds(r0, ROWS_PT)], sem1)
    pltpu.async_copy(src_hbm.at[pl.ds(blk0, BLOCKS)], sidx, sem2)
    pltpu.async_copy(dst_hbm.at[pl.ds(blk0, BLOCKS)], didx, sem3)
    pltpu.make_async_copy(zeros_hbm.at[pl.ds(r0, ROWS_PT)], acc_sh.at[pl.ds(r0, ROWS_PT)], sem0).wait()
    pltpu.make_async_copy(g_hbm.at[pl.ds(r0, ROWS_PT)], g_spm.at[pl.ds(r0, ROWS_PT)], sem1).wait()
    pltpu.make_async_copy(src_hbm.at[pl.ds(blk0, BLOCKS)], sidx, sem2).wait()
    pltpu.make_async_copy(dst_hbm.at[pl.ds(blk0, BLOCKS)], didx, sem3).wait()
    plsc.subcore_barrier()

    pltpu.async_copy(g_spm.at[sidx.at[0]], rows0, sem0)

    def body(j, carry):
        b0 = j * 2
        b1 = b0 + 1
        pltpu.make_async_copy(g_spm.at[sidx.at[b0]], rows0, sem0).wait()
        pltpu.async_copy(g_spm.at[sidx.at[b1]], rows1, sem1)
        pltpu.sync_copy(rows0, acc_sh.at[didx.at[b0]], add=True)
        pltpu.make_async_copy(g_spm.at[sidx.at[b1]], rows1, sem1).wait()

        @pl.when(b1 + 1 < BLOCKS)
        def _():
            pltpu.async_copy(g_spm.at[sidx.at[b1 + 1]], rows0, sem0)

        pltpu.sync_copy(rows1, acc_sh.at[didx.at[b1]], add=True)
        return carry

    lax.fori_loop(0, BLOCKS // 2, body, 0)
    plsc.subcore_barrier()
    pltpu.sync_copy(acc_sh.at[pl.ds(r0, ROWS_PT)], out_hbm.at[c].at[pl.ds(r0, ROWS_PT)])


CH = 128
NCH = ROWS_PT // CH


@functools.partial(
    pl.kernel,
    out_type=[jax.ShapeDtypeStruct((NC, NP, D), jnp.float32),
              jax.ShapeDtypeStruct((NP, D), jnp.float32)],
    mesh=_MESH,
    scratch_types=[
        pltpu.VMEM_SHARED((NP, D), jnp.float32),
        pltpu.VMEM_SHARED((NP, D), jnp.float32),
        pltpu.VMEM((BLOCKS, B), jnp.int32),
        pltpu.VMEM((BLOCKS, B), jnp.int32),
        pltpu.VMEM((B, D), jnp.float32),
        pltpu.VMEM((B, D), jnp.float32),
        pltpu.VMEM((CH, D), jnp.float32),
        pltpu.VMEM((CH, D), jnp.float32),
        pltpu.VMEM((CH, D), jnp.float32),
        pltpu.VMEM((CH, 16), jnp.float32),
        pltpu.VMEM((8, D), jnp.float32),
        pltpu.SemaphoreType.DMA,
        pltpu.SemaphoreType.DMA,
        pltpu.SemaphoreType.DMA,
        pltpu.SemaphoreType.DMA,
    ],
    compiler_params=pltpu.CompilerParams(use_tc_tiling_on_sc=False),
)
def _sc_fused_last(a_hbm, g3_hbm, dinvb_hbm, b_hbm, src_hbm, dst_hbm, zeros_hbm,
                   out_hbm, g4_hbm, acc_sh, g_spm, sidx, didx, rows0, rows1,
                   a0c, a1c, gc, dvc, bc, sem0, sem1, sem2, sem3):
    c = lax.axis_index("c")
    s = lax.axis_index("s")
    r0 = s * ROWS_PT
    blk0 = (c * NS + s) * BLOCKS
    pltpu.async_copy(zeros_hbm.at[pl.ds(r0, ROWS_PT)], acc_sh.at[pl.ds(r0, ROWS_PT)], sem0)
    pltpu.async_copy(src_hbm.at[pl.ds(blk0, BLOCKS)], sidx, sem2)
    pltpu.async_copy(dst_hbm.at[pl.ds(blk0, BLOCKS)], didx, sem3)
    pltpu.sync_copy(b_hbm, bc)
    pltpu.make_async_copy(zeros_hbm.at[pl.ds(r0, ROWS_PT)], acc_sh.at[pl.ds(r0, ROWS_PT)], sem0).wait()
    pltpu.make_async_copy(src_hbm.at[pl.ds(blk0, BLOCKS)], sidx, sem2).wait()
    pltpu.make_async_copy(dst_hbm.at[pl.ds(blk0, BLOCKS)], didx, sem3).wait()
    blo = bc[0, pl.ds(0, 16)]
    bhi = bc[0, pl.ds(16, 16)]

    for ci in range(NCH):
        rc = r0 + ci * CH
        pltpu.async_copy(a_hbm.at[0].at[pl.ds(rc, CH)], a0c, sem0)
        pltpu.async_copy(a_hbm.at[1].at[pl.ds(rc, CH)], a1c, sem1)
        pltpu.async_copy(g3_hbm.at[pl.ds(rc, CH)], gc, sem2)
        pltpu.async_copy(dinvb_hbm.at[pl.ds(rc, CH)], dvc, sem3)
        pltpu.make_async_copy(a_hbm.at[0].at[pl.ds(rc, CH)], a0c, sem0).wait()
        pltpu.make_async_copy(a_hbm.at[1].at[pl.ds(rc, CH)], a1c, sem1).wait()
        pltpu.make_async_copy(g3_hbm.at[pl.ds(rc, CH)], gc, sem2).wait()
        pltpu.make_async_copy(dinvb_hbm.at[pl.ds(rc, CH)], dvc, sem3).wait()

        def row_body(r, carry):
            vd = dvc[r, :]
            lo = a0c[r, pl.ds(0, 16)] + a1c[r, pl.ds(0, 16)] + gc[r, pl.ds(0, 16)]
            hi = a0c[r, pl.ds(16, 16)] + a1c[r, pl.ds(16, 16)] + gc[r, pl.ds(16, 16)]
            a0c[r, pl.ds(0, 16)] = jnp.maximum(lo * vd + blo, 0.0) * vd
            a0c[r, pl.ds(16, 16)] = jnp.maximum(hi * vd + bhi, 0.0) * vd
            return carry

        lax.fori_loop(0, CH, row_body, 0)
        pltpu.sync_copy(a0c, g_spm.at[pl.ds(rc, CH)])

        @pl.when(c == 0)
        def _():
            pltpu.sync_copy(a0c, g4_hbm.at[pl.ds(rc, CH)])

    plsc.subcore_barrier()

    pltpu.async_copy(g_spm.at[sidx.at[0]], rows0, sem0)

    def body(j, carry):
        b0 = j * 2
        b1 = b0 + 1
        pltpu.make_async_copy(g_spm.at[sidx.at[b0]], rows0, sem0).wait()
        pltpu.async_copy(g_spm.at[sidx.at[b1]], rows1, sem1)
        pltpu.sync_copy(rows0, acc_sh.at[didx.at[b0]], add=True)
        pltpu.make_async_copy(g_spm.at[sidx.at[b1]], rows1, sem1).wait()

        @pl.when(b1 + 1 < BLOCKS)
        def _():
            pltpu.async_copy(g_spm.at[sidx.at[b1 + 1]], rows0, sem0)

        pltpu.sync_copy(rows1, acc_sh.at[didx.at[b1]], add=True)
        return carry

    lax.fori_loop(0, BLOCKS // 2, body, 0)
    plsc.subcore_barrier()
    pltpu.sync_copy(acc_sh.at[pl.ds(r0, ROWS_PT)], out_hbm.at[c].at[pl.ds(r0, ROWS_PT)])


def _tc_matmul1(x_ref, w1_ref, h_ref):
    h_ref[...] = jnp.dot(x_ref[...], w1_ref[...],
                         preferred_element_type=jnp.float32)


def _tc_head(degpt_ref, h_ref, dinv_ref, dinvb_ref, g1_ref):
    dp = degpt_ref[...]
    deg = dp[:, 0:1] + dp[:, 1:2] + 1.0
    dinv = lax.rsqrt(deg)
    dinv_ref[...] = dinv
    dinvb_ref[...] = jnp.broadcast_to(dinv, (NP, 16))
    g1_ref[...] = h_ref[...] * dinv


def _tc_mid(a0_ref, a1_ref, g_ref, dinv_ref, b_ref, w_ref, gn_ref):
    dinv = dinv_ref[...]
    agg = (a0_ref[...] + a1_ref[...] + g_ref[...]) * dinv + b_ref[...]
    out = jnp.maximum(agg, 0.0)
    gn_ref[...] = jnp.dot(out, w_ref[...], preferred_element_type=jnp.float32) * dinv


def _tc_tail(a0_ref, a1_ref, g_ref, dinv_ref, w4_ref, b4_ref, out_ref):
    a = (a0_ref[...] + a1_ref[...] + g_ref[...]) * dinv_ref[...]
    logits = jnp.dot(a, w4_ref[...], preferred_element_type=jnp.float32) + b4_ref[...]
    m = jnp.max(logits, axis=1, keepdims=True)
    z = logits - m
    out_ref[...] = z - jnp.log(jnp.sum(jnp.exp(z), axis=1, keepdims=True))


def _call(body, out_shapes, *args):
    return pl.pallas_call(
        body,
        out_shape=[jax.ShapeDtypeStruct(s, jnp.float32) for s in out_shapes],
    )(*args)


@jax.jit
def kernel(x, edge_index, W1, b1, W2, b2, W3, b3, W4, b4):
    src = edge_index[0].astype(jnp.int32)
    dst = edge_index[1].astype(jnp.int32)
    pad = EP - E
    src_p = jnp.concatenate([src, jnp.zeros((pad,), jnp.int32)])
    dst_p = jnp.concatenate([dst, jnp.full((pad,), TRASH, jnp.int32)])
    src2d = src_p.reshape(EP // B, B)
    dst2d = dst_p.reshape(EP // B, B)
    zeros1 = jnp.zeros((NP,), jnp.float32)
    zeros2 = jnp.zeros((NP, D), jnp.float32)

    degp = _sc_degree(dst_p.reshape(EP // BD, BD), zeros1)
    (h1,) = _call(_tc_matmul1, [(N_NODES, D)], x, W1)
    h1_p = jnp.pad(h1, ((0, NP - N_NODES), (0, 0)))
    dinv, dinvb, g = _call(_tc_head, [(NP, 1), (NP, 16), (NP, D)], degp.T, h1_p)

    for bk, wn in ((b1, W2), (b2, W3)):
        ap = _sc_aggregate(g, src2d, dst2d, zeros2)
        (g,) = _call(_tc_mid, [(NP, D)], ap[0], ap[1], g, dinv,
                     bk.reshape(1, D), wn)

    ap = _sc_aggregate(g, src2d, dst2d, zeros2)
    b3b = jnp.broadcast_to(b3.reshape(1, D), (8, D))
    ap4, g4 = _sc_fused_last(ap, g, dinvb, b3b, src2d, dst2d, zeros2)
    (out,) = _call(_tc_tail, [(NP, 2)], ap4[0], ap4[1], g4, dinv, W4,
                   b4.reshape(1, 2))
    return out[:N_NODES]


# speedup vs baseline: 1.0666x; 1.0061x over previous
import functools

import jax
import jax.numpy as jnp
from jax import lax
from jax.experimental import pallas as pl
from jax.experimental.pallas import tpu as pltpu
from jax.experimental.pallas import tpu_sc as plsc

N_NODES = 10000
NP = 10240
D = 32
E = 320000
NC, NS, B = 2, 16, 256
BD = 128
EP = 327680
BLOCKS = EP // (NC * NS * B)
BLOCKS_D = EP // (NC * NS * BD)
ROWS_PT = NP // NS
TRASH = N_NODES

_MESH = plsc.VectorSubcoreMesh(
    core_axis_name="c", subcore_axis_name="s", num_cores=NC, num_subcores=NS)


@functools.partial(
    pl.kernel,
    out_type=jax.ShapeDtypeStruct((NC, NP), jnp.float32),
    mesh=_MESH,
    scratch_types=[
        pltpu.VMEM_SHARED((NP,), jnp.float32),
        pltpu.VMEM((BLOCKS_D, BD), jnp.int32),
        pltpu.VMEM((BD,), jnp.float32),
        pltpu.SemaphoreType.DMA,
        pltpu.SemaphoreType.DMA,
    ],
)
def _sc_degree(dst_hbm, zeros_hbm, out_hbm, acc_sh, didx, ones_v, sem0, sem1):
    c = lax.axis_index("c")
    s = lax.axis_index("s")
    r0 = s * ROWS_PT
    blk0 = (c * NS + s) * BLOCKS_D
    pltpu.async_copy(zeros_hbm.at[pl.ds(r0, ROWS_PT)], acc_sh.at[pl.ds(r0, ROWS_PT)], sem0)
    pltpu.async_copy(dst_hbm.at[pl.ds(blk0, BLOCKS_D)], didx, sem1)
    for i in range(BD // 16):
        ones_v[pl.ds(i * 16, 16)] = jnp.full((16,), 1.0, jnp.float32)
    pltpu.make_async_copy(zeros_hbm.at[pl.ds(r0, ROWS_PT)], acc_sh.at[pl.ds(r0, ROWS_PT)], sem0).wait()
    pltpu.make_async_copy(dst_hbm.at[pl.ds(blk0, BLOCKS_D)], didx, sem1).wait()
    plsc.subcore_barrier()

    def body(j, carry):
        pltpu.sync_copy(ones_v, acc_sh.at[didx.at[j]], add=True)
        return carry

    lax.fori_loop(0, BLOCKS_D, body, 0)
    plsc.subcore_barrier()
    pltpu.sync_copy(acc_sh.at[pl.ds(r0, ROWS_PT)], out_hbm.at[c].at[pl.ds(r0, ROWS_PT)])


@functools.partial(
    pl.kernel,
    out_type=jax.ShapeDtypeStruct((NC, NP, D), jnp.float32),
    mesh=_MESH,
    scratch_types=[
        pltpu.VMEM_SHARED((NP, D), jnp.float32),
        pltpu.VMEM_SHARED((NP, D), jnp.float32),
        pltpu.VMEM((BLOCKS, B), jnp.int32),
        pltpu.VMEM((BLOCKS, B), jnp.int32),
        pltpu.VMEM((B, D), jnp.float32),
        pltpu.VMEM((B, D), jnp.float32),
        pltpu.SemaphoreType.DMA,
        pltpu.SemaphoreType.DMA,
        pltpu.SemaphoreType.DMA,
        pltpu.SemaphoreType.DMA,
    ],
    compiler_params=pltpu.CompilerParams(use_tc_tiling_on_sc=False),
)
def _sc_aggregate(g_hbm, src_hbm, dst_hbm, zeros_hbm, out_hbm,
                  acc_sh, g_spm, sidx, didx, rows0, rows1, sem0, sem1,
                  sem2, sem3):
    c = lax.axis_index("c")
    s = lax.axis_index("s")
    r0 = s * ROWS_PT
    blk0 = (c * NS + s) * BLOCKS
    pltpu.async_copy(zeros_hbm.at[pl.ds(r0, ROWS_PT)], acc_sh.at[pl.ds(r0, ROWS_PT)], sem0)
    pltpu.async_copy(g_hbm.at[pl.ds(r0, ROWS_PT)], g_spm.at[pl.ds(r0, ROWS_PT)], sem1)
    pltpu.async_copy(src_hbm.at[pl.ds(blk0, BLOCKS)], sidx, sem2)
    pltpu.async_copy(dst_hbm.at[pl.ds(blk0, BLOCKS)], didx, sem3)
    pltpu.make_async_copy(zeros_hbm.at[pl.ds(r0, ROWS_PT)], acc_sh.at[pl.ds(r0, ROWS_PT)], sem0).wait()
    pltpu.make_async_copy(g_hbm.at[pl.ds(r0, ROWS_PT)], g_spm.at[pl.ds(r0, ROWS_PT)], sem1).wait()
    pltpu.make_async_copy(src_hbm.at[pl.ds(blk0, BLOCKS)], sidx, sem2).wait()
    pltpu.make_async_copy(dst_hbm.at[pl.ds(blk0, BLOCKS)], didx, sem3).wait()
    plsc.subcore_barrier()

    pltpu.async_copy(g_spm.at[sidx.at[0]], rows0, sem0)

    def body(j, carry):
        b0 = j * 2
        b1 = b0 + 1
        pltpu.make_async_copy(g_spm.at[sidx.at[b0]], rows0, sem0).wait()
        pltpu.async_copy(g_spm.at[sidx.at[b1]], rows1, sem1)
        pltpu.sync_copy(rows0, acc_sh.at[didx.at[b0]], add=True)
        pltpu.make_async_copy(g_spm.at[sidx.at[b1]], rows1, sem1).wait()

        @pl.when(b1 + 1 < BLOCKS)
        def _():
            pltpu.async_copy(g_spm.at[sidx.at[b1 + 1]], rows0, sem0)

        pltpu.sync_copy(rows1, acc_sh.at[didx.at[b1]], add=True)
        return carry

    lax.fori_loop(0, BLOCKS // 2, body, 0)
    plsc.subcore_barrier()
    pltpu.sync_copy(acc_sh.at[pl.ds(r0, ROWS_PT)], out_hbm.at[c].at[pl.ds(r0, ROWS_PT)])


CH = 320
NCH = ROWS_PT // CH


@functools.partial(
    pl.kernel,
    out_type=[jax.ShapeDtypeStruct((NC, NP, D), jnp.float32),
              jax.ShapeDtypeStruct((NP, D), jnp.float32)],
    mesh=_MESH,
    scratch_types=[
        pltpu.VMEM_SHARED((NP, D), jnp.float32),
        pltpu.VMEM_SHARED((NP, D), jnp.float32),
        pltpu.VMEM((BLOCKS, B), jnp.int32),
        pltpu.VMEM((BLOCKS, B), jnp.int32),
        pltpu.VMEM((B, D), jnp.float32),
        pltpu.VMEM((B, D), jnp.float32),
        pltpu.VMEM((CH, D), jnp.float32),
        pltpu.VMEM((CH, D), jnp.float32),
        pltpu.VMEM((CH, D), jnp.float32),
        pltpu.VMEM((CH, 16), jnp.float32),
        pltpu.VMEM((8, D), jnp.float32),
        pltpu.SemaphoreType.DMA,
        pltpu.SemaphoreType.DMA,
        pltpu.SemaphoreType.DMA,
        pltpu.SemaphoreType.DMA,
    ],
    compiler_params=pltpu.CompilerParams(use_tc_tiling_on_sc=False),
)
def _sc_fused_last(a_hbm, g3_hbm, dinvb_hbm, b_hbm, src_hbm, dst_hbm, zeros_hbm,
                   out_hbm, g4_hbm, acc_sh, g_spm, sidx, didx, rows0, rows1,
                   a0c, a1c, gc, dvc, bc, sem0, sem1, sem2, sem3):
    c = lax.axis_index("c")
    s = lax.axis_index("s")
    r0 = s * ROWS_PT
    blk0 = (c * NS + s) * BLOCKS
    pltpu.async_copy(zeros_hbm.at[pl.ds(r0, ROWS_PT)], acc_sh.at[pl.ds(r0, ROWS_PT)], sem0)
    pltpu.async_copy(src_hbm.at[pl.ds(blk0, BLOCKS)], sidx, sem2)
    pltpu.async_copy(dst_hbm.at[pl.ds(blk0, BLOCKS)], didx, sem3)
    pltpu.sync_copy(b_hbm, bc)
    pltpu.make_async_copy(zeros_hbm.at[pl.ds(r0, ROWS_PT)], acc_sh.at[pl.ds(r0, ROWS_PT)], sem0).wait()
    pltpu.make_async_copy(src_hbm.at[pl.ds(blk0, BLOCKS)], sidx, sem2).wait()
    pltpu.make_async_copy(dst_hbm.at[pl.ds(blk0, BLOCKS)], didx, sem3).wait()
    blo = bc[0, pl.ds(0, 16)]
    bhi = bc[0, pl.ds(16, 16)]

    for ci in range(NCH):
        rc = r0 + ci * CH
        pltpu.async_copy(a_hbm.at[0].at[pl.ds(rc, CH)], a0c, sem0)
        pltpu.async_copy(a_hbm.at[1].at[pl.ds(rc, CH)], a1c, sem1)
        pltpu.async_copy(g3_hbm.at[pl.ds(rc, CH)], gc, sem2)
        pltpu.async_copy(dinvb_hbm.at[pl.ds(rc, CH)], dvc, sem3)
        pltpu.make_async_copy(a_hbm.at[0].at[pl.ds(rc, CH)], a0c, sem0).wait()
        pltpu.make_async_copy(a_hbm.at[1].at[pl.ds(rc, CH)], a1c, sem1).wait()
        pltpu.make_async_copy(g3_hbm.at[pl.ds(rc, CH)], gc, sem2).wait()
        pltpu.make_async_copy(dinvb_hbm.at[pl.ds(rc, CH)], dvc, sem3).wait()

        def row_body(r, carry):
            vd = dvc[r, :]
            lo = a0c[r, pl.ds(0, 16)] + a1c[r, pl.ds(0, 16)] + gc[r, pl.ds(0, 16)]
            hi = a0c[r, pl.ds(16, 16)] + a1c[r, pl.ds(16, 16)] + gc[r, pl.ds(16, 16)]
            a0c[r, pl.ds(0, 16)] = jnp.maximum(lo * vd + blo, 0.0) * vd
            a0c[r, pl.ds(16, 16)] = jnp.maximum(hi * vd + bhi, 0.0) * vd
            return carry

        lax.fori_loop(0, CH, row_body, 0)
        pltpu.sync_copy(a0c, g_spm.at[pl.ds(rc, CH)])

        @pl.when(c == 0)
        def _():
            pltpu.sync_copy(a0c, g4_hbm.at[pl.ds(rc, CH)])

    plsc.subcore_barrier()

    pltpu.async_copy(g_spm.at[sidx.at[0]], rows0, sem0)

    def body(j, carry):
        b0 = j * 2
        b1 = b0 + 1
        pltpu.make_async_copy(g_spm.at[sidx.at[b0]], rows0, sem0).wait()
        pltpu.async_copy(g_spm.at[sidx.at[b1]], rows1, sem1)
        pltpu.sync_copy(rows0, acc_sh.at[didx.at[b0]], add=True)
        pltpu.make_async_copy(g_spm.at[sidx.at[b1]], rows1, sem1).wait()

        @pl.when(b1 + 1 < BLOCKS)
        def _():
            pltpu.async_copy(g_spm.at[sidx.at[b1 + 1]], rows0, sem0)

        pltpu.sync_copy(rows1, acc_sh.at[didx.at[b1]], add=True)
        return carry

    lax.fori_loop(0, BLOCKS // 2, body, 0)
    plsc.subcore_barrier()
    pltpu.sync_copy(acc_sh.at[pl.ds(r0, ROWS_PT)], out_hbm.at[c].at[pl.ds(r0, ROWS_PT)])


def _tc_matmul1(x_ref, w1_ref, h_ref):
    h_ref[...] = jnp.dot(x_ref[...], w1_ref[...],
                         preferred_element_type=jnp.float32)


def _tc_head(degpt_ref, h_ref, dinv_ref, dinvb_ref, g1_ref):
    dp = degpt_ref[...]
    deg = dp[:, 0:1] + dp[:, 1:2] + 1.0
    dinv = lax.rsqrt(deg)
    dinv_ref[...] = dinv
    dinvb_ref[...] = jnp.broadcast_to(dinv, (NP, 16))
    g1_ref[...] = h_ref[...] * dinv


def _tc_mid(a0_ref, a1_ref, g_ref, dinv_ref, b_ref, w_ref, gn_ref):
    dinv = dinv_ref[...]
    agg = (a0_ref[...] + a1_ref[...] + g_ref[...]) * dinv + b_ref[...]
    out = jnp.maximum(agg, 0.0)
    gn_ref[...] = jnp.dot(out, w_ref[...], preferred_element_type=jnp.float32) * dinv


def _tc_tail(a0_ref, a1_ref, g_ref, dinv_ref, w4_ref, b4_ref, out_ref):
    a = (a0_ref[...] + a1_ref[...] + g_ref[...]) * dinv_ref[...]
    logits = jnp.dot(a, w4_ref[...], preferred_element_type=jnp.float32) + b4_ref[...]
    m = jnp.max(logits, axis=1, keepdims=True)
    z = logits - m
    out_ref[...] = z - jnp.log(jnp.sum(jnp.exp(z), axis=1, keepdims=True))


def _call(body, out_shapes, *args):
    return pl.pallas_call(
        body,
        out_shape=[jax.ShapeDtypeStruct(s, jnp.float32) for s in out_shapes],
    )(*args)


@jax.jit
def kernel(x, edge_index, W1, b1, W2, b2, W3, b3, W4, b4):
    src = edge_index[0].astype(jnp.int32)
    dst = edge_index[1].astype(jnp.int32)
    pad = EP - E
    src_p = jnp.concatenate([src, jnp.zeros((pad,), jnp.int32)])
    dst_p = jnp.concatenate([dst, jnp.full((pad,), TRASH, jnp.int32)])
    src2d = src_p.reshape(EP // B, B)
    dst2d = dst_p.reshape(EP // B, B)
    zeros1 = jnp.zeros((NP,), jnp.float32)
    zeros2 = jnp.zeros((NP, D), jnp.float32)

    degp = _sc_degree(dst_p.reshape(EP // BD, BD), zeros1)
    (h1,) = _call(_tc_matmul1, [(N_NODES, D)], x, W1)
    h1_p = jnp.pad(h1, ((0, NP - N_NODES), (0, 0)))
    dinv, dinvb, g = _call(_tc_head, [(NP, 1), (NP, 16), (NP, D)], degp.T, h1_p)

    for bk, wn in ((b1, W2), (b2, W3)):
        ap = _sc_aggregate(g, src2d, dst2d, zeros2)
        (g,) = _call(_tc_mid, [(NP, D)], ap[0], ap[1], g, dinv,
                     bk.reshape(1, D), wn)

    ap = _sc_aggregate(g, src2d, dst2d, zeros2)
    b3b = jnp.broadcast_to(b3.reshape(1, D), (8, D))
    ap4, g4 = _sc_fused_last(ap, g, dinvb, b3b, src2d, dst2d, zeros2)
    (out,) = _call(_tc_tail, [(NP, 2)], ap4[0], ap4[1], g4, dinv, W4,
                   b4.reshape(1, 2))
    return out[:N_NODES]
